# Initial kernel scaffold; baseline (speedup 1.0000x reference)
import jax
import jax.numpy as jnp
from jax.experimental import pallas as pl


def kernel(user_features, W):
    raise NotImplementedError("write your pallas kernel here")


# speedup vs baseline: 1.9061x; 1.9061x over previous
import functools

import jax
import jax.numpy as jnp
from jax.experimental import pallas as pl

HID = 64
KCAP = 8
KMIN = 2
KMAX = 8
NITER = 3
SEQ = 50


def _softmax(b):
    m = jnp.max(b, axis=-1, keepdims=True)
    e = jnp.exp(b - m)
    return e / jnp.sum(e, axis=-1, keepdims=True)


def _squash(caps):
    cn = jnp.sum(caps * caps, axis=-1, keepdims=True)
    return caps * (cn / (1.0 + cn) / jnp.sqrt(cn + 1e-9))


def _ckl_blh(c, mat):
    cols = [jnp.sum(c[:, k, :, None] * mat, axis=1) for k in range(KCAP)]
    return jnp.stack(cols, axis=1)


def _blh_bkh(mat, caps):
    rows = [jnp.sum(mat * caps[:, k, None, :], axis=2) for k in range(KCAP)]
    return jnp.stack(rows, axis=1)


def _block_kernel(x_ref, wt_ref, b0_ref, out_ref, *, bb):
    x = x_ref[...]
    wt = wt_ref[...]
    lin = jnp.dot(
        x.reshape(bb * SEQ, HID), wt, preferred_element_type=jnp.float32
    ).reshape(bb, SEQ, HID)

    b = b0_ref[...]
    c = None
    caps = None
    for t in range(NITER):
        c = _softmax(b)
        caps = _squash(_ckl_blh(c, lin))
        if t < NITER - 1:
            b = b + _blh_bkh(lin, caps)

    strength = jnp.sqrt(jnp.sum(caps * caps, axis=-1))

    e_nrm = jnp.sqrt(jnp.sum(x * x, axis=-1, keepdims=True))
    e_norm = x / (e_nrm + 1e-8)

    csum = jnp.sum(c, axis=-1)
    mu = jax.lax.dot_general(
        c, x, (((2,), (1,)), ((0,), (0,))),
        preferred_element_type=jnp.float32,
    ) / (csum[:, :, None] + 1e-8)
    mu_nrm = jnp.sqrt(jnp.sum(mu * mu, axis=-1, keepdims=True))
    mu_norm = mu / (mu_nrm + 1e-8)

    cos = jax.lax.dot_general(
        mu_norm, e_norm, (((2,), (2,)), ((0,), (0,))),
        preferred_element_type=jnp.float32,
    )
    s = jnp.sum(c * cos, axis=-1)
    pair = jax.lax.dot_general(
        mu_norm, mu_norm, (((2,), (2,)), ((0,), (0,))),
        preferred_element_type=jnp.float32,
    )

    si = strength[:, :, None]
    sj = strength[:, None, :]
    ii = jax.lax.broadcasted_iota(jnp.int32, (KCAP, KCAP), 0)
    jj = jax.lax.broadcasted_iota(jnp.int32, (KCAP, KCAP), 1)
    beats = (si > sj) | ((si == sj) & (ii < jj)[None, :, :])
    rank = jnp.sum(beats.astype(jnp.int32), axis=1)
    offdiag = (ii != jj).astype(jnp.float32)[None, :, :]

    score_cols = []
    for k in range(KMIN, KMAX + 1):
        maskk = (rank < k).astype(jnp.float32)
        cons = jnp.sum(maskk * s, axis=-1) / float(k * SEQ)
        pmask = maskk[:, :, None] * maskk[:, None, :] * offdiag
        psum = 0.5 * jnp.sum(pmask * pair, axis=(1, 2))
        div = 1.0 - (2.0 / float(k * (k - 1))) * psum
        score_cols.append(0.5 * cons + 0.5 * div)
    scores = jnp.stack(score_cols, axis=1)

    opt = jnp.argmax(scores, axis=1).astype(jnp.float32) + float(KMIN)
    out_ref[...] = jnp.concatenate([scores, opt[:, None]], axis=1)


@jax.jit
def kernel(user_features, W):
    bsz, seq, hid = user_features.shape
    bb = 32
    grid = (bsz // bb,)
    b0 = jax.random.normal(jax.random.key(1), (bsz, KCAP, seq), dtype=jnp.float32)
    wt = W.T

    out = pl.pallas_call(
        functools.partial(_block_kernel, bb=bb),
        grid=grid,
        in_specs=[
            pl.BlockSpec((bb, seq, hid), lambda i: (i, 0, 0)),
            pl.BlockSpec((hid, hid), lambda i: (0, 0)),
            pl.BlockSpec((bb, KCAP, seq), lambda i: (i, 0, 0)),
        ],
        out_specs=pl.BlockSpec((bb, KMAX - KMIN + 2), lambda i: (i, 0)),
        out_shape=jax.ShapeDtypeStruct((bsz, KMAX - KMIN + 2), jnp.float32),
    )(user_features, wt, b0)

    k_scores = out[:, : KMAX - KMIN + 1]
    optimal_k = out[:, KMAX - KMIN + 1].astype(jnp.int32)
    return (optimal_k, k_scores)


# speedup vs baseline: 3.6210x; 1.8997x over previous
import functools

import jax
import jax.numpy as jnp
from jax.experimental import pallas as pl

HID = 64
KCAP = 8
KMIN = 2
KMAX = 8
NITER = 3
SEQ = 50


def _softmax(b):
    m = jnp.max(b, axis=-1, keepdims=True)
    e = jnp.exp(b - m)
    return e / jnp.sum(e, axis=-1, keepdims=True)


def _squash(caps):
    cn = jnp.sum(caps * caps, axis=-1, keepdims=True)
    return caps * (cn / (1.0 + cn) / jnp.sqrt(cn + 1e-9))


def _ckl_blh(c, mat):
    return jax.lax.dot_general(
        c, mat, (((2,), (1,)), ((0,), (0,))),
        preferred_element_type=jnp.float32,
        precision=jax.lax.Precision.HIGHEST,
    )


def _blh_bkh(mat, caps):
    return jax.lax.dot_general(
        caps, mat, (((2,), (2,)), ((0,), (0,))),
        preferred_element_type=jnp.float32,
        precision=jax.lax.Precision.HIGHEST,
    )


def _block_kernel(x_ref, wt_ref, b0_ref, out_ref, *, bb):
    x = x_ref[...]
    wt = wt_ref[...]
    lin = jnp.dot(
        x.reshape(bb * SEQ, HID), wt, preferred_element_type=jnp.float32
    ).reshape(bb, SEQ, HID)

    b = b0_ref[...]
    c = None
    caps = None
    for t in range(NITER):
        c = _softmax(b)
        caps = _squash(_ckl_blh(c, lin))
        if t < NITER - 1:
            b = b + _blh_bkh(lin, caps)

    strength = jnp.sqrt(jnp.sum(caps * caps, axis=-1))

    e_nrm = jnp.sqrt(jnp.sum(x * x, axis=-1, keepdims=True))
    e_norm = x / (e_nrm + 1e-8)

    csum = jnp.sum(c, axis=-1)
    mu = jax.lax.dot_general(
        c, x, (((2,), (1,)), ((0,), (0,))),
        preferred_element_type=jnp.float32,
    ) / (csum[:, :, None] + 1e-8)
    mu_nrm = jnp.sqrt(jnp.sum(mu * mu, axis=-1, keepdims=True))
    mu_norm = mu / (mu_nrm + 1e-8)

    cos = jax.lax.dot_general(
        mu_norm, e_norm, (((2,), (2,)), ((0,), (0,))),
        preferred_element_type=jnp.float32,
    )
    s = jnp.sum(c * cos, axis=-1)
    pair = jax.lax.dot_general(
        mu_norm, mu_norm, (((2,), (2,)), ((0,), (0,))),
        preferred_element_type=jnp.float32,
    )

    si = strength[:, :, None]
    sj = strength[:, None, :]
    ii = jax.lax.broadcasted_iota(jnp.int32, (KCAP, KCAP), 0)
    jj = jax.lax.broadcasted_iota(jnp.int32, (KCAP, KCAP), 1)
    beats = (si > sj) | ((si == sj) & (ii < jj)[None, :, :])
    rank = jnp.sum(beats.astype(jnp.int32), axis=1)
    offdiag = (ii != jj).astype(jnp.float32)[None, :, :]

    score_cols = []
    for k in range(KMIN, KMAX + 1):
        maskk = (rank < k).astype(jnp.float32)
        cons = jnp.sum(maskk * s, axis=-1) / float(k * SEQ)
        pmask = maskk[:, :, None] * maskk[:, None, :] * offdiag
        psum = 0.5 * jnp.sum(pmask * pair, axis=(1, 2))
        div = 1.0 - (2.0 / float(k * (k - 1))) * psum
        score_cols.append(0.5 * cons + 0.5 * div)
    scores = jnp.stack(score_cols, axis=1)

    opt = jnp.argmax(scores, axis=1).astype(jnp.float32) + float(KMIN)
    out_ref[...] = jnp.concatenate([scores, opt[:, None]], axis=1)


@jax.jit
def kernel(user_features, W):
    bsz, seq, hid = user_features.shape
    bb = 32
    grid = (bsz // bb,)
    b0 = jax.random.normal(jax.random.key(1), (bsz, KCAP, seq), dtype=jnp.float32)
    wt = W.T

    out = pl.pallas_call(
        functools.partial(_block_kernel, bb=bb),
        grid=grid,
        in_specs=[
            pl.BlockSpec((bb, seq, hid), lambda i: (i, 0, 0)),
            pl.BlockSpec((hid, hid), lambda i: (0, 0)),
            pl.BlockSpec((bb, KCAP, seq), lambda i: (i, 0, 0)),
        ],
        out_specs=pl.BlockSpec((bb, KMAX - KMIN + 2), lambda i: (i, 0)),
        out_shape=jax.ShapeDtypeStruct((bsz, KMAX - KMIN + 2), jnp.float32),
    )(user_features, wt, b0)

    k_scores = out[:, : KMAX - KMIN + 1]
    optimal_k = out[:, KMAX - KMIN + 1].astype(jnp.int32)
    return (optimal_k, k_scores)


# speedup vs baseline: 3.7160x; 1.0262x over previous
import functools

import jax
import jax.numpy as jnp
from jax.experimental import pallas as pl

HID = 64
KCAP = 8
KMIN = 2
KMAX = 8
NITER = 3
SEQ = 50


def _softmax(b):
    m = jnp.max(b, axis=-1, keepdims=True)
    e = jnp.exp(b - m)
    return e / jnp.sum(e, axis=-1, keepdims=True)


def _squash(caps):
    cn = jnp.sum(caps * caps, axis=-1, keepdims=True)
    return caps * (cn / (1.0 + cn) / jnp.sqrt(cn + 1e-9))


def _ckl_blh(c, mat):
    return jax.lax.dot_general(
        c, mat, (((2,), (1,)), ((0,), (0,))),
        preferred_element_type=jnp.float32,
        precision=jax.lax.Precision.HIGHEST,
    )


def _blh_bkh(mat, caps):
    return jax.lax.dot_general(
        caps, mat, (((2,), (2,)), ((0,), (0,))),
        preferred_element_type=jnp.float32,
        precision=jax.lax.Precision.HIGHEST,
    )


def _block_kernel(x_ref, wt_ref, b0_ref, out_ref, *, bb):
    x = x_ref[...]
    wt = wt_ref[...]
    lin = jnp.dot(
        x.reshape(bb * SEQ, HID), wt, preferred_element_type=jnp.float32
    ).reshape(bb, SEQ, HID)

    b = b0_ref[...]
    c = None
    caps = None
    for t in range(NITER):
        c = _softmax(b)
        caps = _squash(_ckl_blh(c, lin))
        if t < NITER - 1:
            b = b + _blh_bkh(lin, caps)

    strength = jnp.sqrt(jnp.sum(caps * caps, axis=-1))

    e_nrm = jnp.sqrt(jnp.sum(x * x, axis=-1, keepdims=True))
    e_norm = x / (e_nrm + 1e-8)

    csum = jnp.sum(c, axis=-1)
    mu = jax.lax.dot_general(
        c, x, (((2,), (1,)), ((0,), (0,))),
        preferred_element_type=jnp.float32,
    ) / (csum[:, :, None] + 1e-8)
    mu_nrm = jnp.sqrt(jnp.sum(mu * mu, axis=-1, keepdims=True))
    mu_norm = mu / (mu_nrm + 1e-8)

    cos = jax.lax.dot_general(
        mu_norm, e_norm, (((2,), (2,)), ((0,), (0,))),
        preferred_element_type=jnp.float32,
    )
    s = jnp.sum(c * cos, axis=-1)
    pair = jax.lax.dot_general(
        mu_norm, mu_norm, (((2,), (2,)), ((0,), (0,))),
        preferred_element_type=jnp.float32,
    )

    si = strength[:, :, None]
    sj = strength[:, None, :]
    ii = jax.lax.broadcasted_iota(jnp.int32, (KCAP, KCAP), 0)
    jj = jax.lax.broadcasted_iota(jnp.int32, (KCAP, KCAP), 1)
    beats = (si > sj) | ((si == sj) & (ii < jj)[None, :, :])
    rank = jnp.sum(beats.astype(jnp.int32), axis=1)
    offdiag = (ii != jj).astype(jnp.float32)[None, :, :]

    score_cols = []
    for k in range(KMIN, KMAX + 1):
        maskk = (rank < k).astype(jnp.float32)
        cons = jnp.sum(maskk * s, axis=-1) / float(k * SEQ)
        pmask = maskk[:, :, None] * maskk[:, None, :] * offdiag
        psum = 0.5 * jnp.sum(pmask * pair, axis=(1, 2))
        div = 1.0 - (2.0 / float(k * (k - 1))) * psum
        score_cols.append(0.5 * cons + 0.5 * div)
    scores = jnp.stack(score_cols, axis=1)

    opt = jnp.argmax(scores, axis=1).astype(jnp.float32) + float(KMIN)
    out_ref[...] = jnp.concatenate([scores, opt[:, None]], axis=1)


@jax.jit
def kernel(user_features, W):
    bsz, seq, hid = user_features.shape
    bb = 64
    grid = (bsz // bb,)
    b0 = jax.random.normal(jax.random.key(1), (bsz, KCAP, seq), dtype=jnp.float32)
    wt = W.T

    out = pl.pallas_call(
        functools.partial(_block_kernel, bb=bb),
        grid=grid,
        in_specs=[
            pl.BlockSpec((bb, seq, hid), lambda i: (i, 0, 0)),
            pl.BlockSpec((hid, hid), lambda i: (0, 0)),
            pl.BlockSpec((bb, KCAP, seq), lambda i: (i, 0, 0)),
        ],
        out_specs=pl.BlockSpec((bb, KMAX - KMIN + 2), lambda i: (i, 0)),
        out_shape=jax.ShapeDtypeStruct((bsz, KMAX - KMIN + 2), jnp.float32),
    )(user_features, wt, b0)

    k_scores = out[:, : KMAX - KMIN + 1]
    optimal_k = out[:, KMAX - KMIN + 1].astype(jnp.int32)
    return (optimal_k, k_scores)
